# Initial kernel scaffold; baseline (speedup 1.0000x reference)
import jax
import jax.numpy as jnp
from jax.experimental import pallas as pl


def kernel(loc_data, conf_data, landm_pca_weight, prior_landmarks, targets, pca_mean, pca_features, singular_values):
    raise NotImplementedError("write your pallas kernel here")


# speedup vs baseline: 71.9854x; 71.9854x over previous
import jax
import jax.numpy as jnp
from jax import lax
from jax.experimental import pallas as pl

_B, _P, _G = 16, 16384, 32
_K = 32
_LDIM = 8
_SCALE = 640.0
_NEG_POS_RATIO = 7
_THRESH = 500.0
_PT = 2048
_T = _P // _PT


def _phase_a(conf_ref, lpw_ref, priors_ref, targets_ref, pm_ref, feat_ref,
             sv_ref, x_ref, acc_ref):
    t = pl.program_id(0)
    b = pl.program_id(1)

    priors_sc = priors_ref[...] * _SCALE
    tg = targets_ref[0]
    gt = tg[:, 4:4 + _LDIM] * _SCALE
    labels_col = tg[:, 12:13]

    pn_row = jnp.sum(priors_sc * priors_sc, axis=0, keepdims=True)
    gn_col = jnp.sum(gt * gt, axis=1, keepdims=True)
    cross = jnp.dot(gt, priors_sc, preferred_element_type=jnp.float32)
    d2 = pn_row + gn_col - 2.0 * cross
    d2c = jnp.maximum(d2, 0.0)
    dmin = jnp.min(d2c, axis=0, keepdims=True)
    gidx = lax.broadcasted_iota(jnp.int32, (_G, _PT), 0)
    cand = jnp.where(d2c == dmin, gidx, _G)
    bestg = jnp.min(cand, axis=0, keepdims=True)
    onehot = (gidx == bestg).astype(jnp.float32)

    matched = lax.dot_general(gt, onehot, (((0,), (0,)), ((), ())),
                              preferred_element_type=jnp.float32)
    mlabel = jnp.sum(onehot * labels_col, axis=0, keepdims=True)
    best_d = jnp.sqrt(dmin)
    posb = (best_d < _THRESH) & (mlabel > 0.0)
    posf = posb.astype(jnp.float32)
    npos_t = jnp.sum(posf)

    residual = matched - priors_sc - pm_ref[...]
    wt = lax.dot_general(residual, feat_ref[...], (((0,), (1,)), ((), ())),
                         preferred_element_type=jnp.float32)
    lpw = lpw_ref[0]
    ad = jnp.abs(lpw - wt)
    sl1 = jnp.where(ad < 1.0, 0.5 * ad * ad, ad - 0.5)
    s_col = jnp.sum(sl1, axis=1, keepdims=True)
    wpen = jnp.sqrt(lpw * lpw / sv_ref[...])
    w_col = jnp.sum(wpen, axis=1, keepdims=True)
    sl1v = jnp.dot(posf, s_col, preferred_element_type=jnp.float32)[0, 0]
    wv = jnp.dot(posf, w_col, preferred_element_type=jnp.float32)[0, 0]

    c = conf_ref[0]
    c0 = c[0:1, :]
    c1 = c[1:2, :]
    m = jnp.maximum(c0, c1)
    lse = m + jnp.log(jnp.exp(c0 - m) + jnp.exp(c1 - m))
    cep = jnp.sum(jnp.where(posb, lse - c1, 0.0))
    x = jnp.where(posb, 0.0, lse - c0)
    x_ref[...] = x.reshape(1, 1, _PT)

    ri = lax.broadcasted_iota(jnp.int32, (_B, 128), 0)
    li = lax.broadcasted_iota(jnp.int32, (_B, 128), 1)
    vals = jnp.where(li == 0, npos_t,
                     jnp.where(li == 1, sl1v,
                               jnp.where(li == 2, wv,
                                         jnp.where(li == 3, cep, 0.0))))
    contrib = jnp.where(ri == b, vals, 0.0)
    first = (t == 0) & (b == 0)

    @pl.when(first)
    def _():
        acc_ref[...] = contrib

    @pl.when(jnp.logical_not(first))
    def _():
        acc_ref[...] = acc_ref[...] + contrib


def _phase_b(x_ref, acc_ref, out_ref):
    x = x_ref[:, 0, :]
    acc = acc_ref[...]
    npos_col = acc[:, 0:1]
    sl1_sum = jnp.sum(acc[:, 1:2])
    wpen_sum = jnp.sum(acc[:, 2:3])
    cepos_sum = jnp.sum(acc[:, 3:4])

    npos_i = npos_col.astype(jnp.int32)
    jcol = jnp.minimum(_NEG_POS_RATIO * npos_i, _P - npos_i)
    bits = lax.bitcast_convert_type(x, jnp.int32)

    def body(i, tcur):
        bit = 30 - i
        trial = jnp.bitwise_or(tcur, jnp.left_shift(jnp.int32(1), bit))
        cnt = jnp.sum((bits >= trial).astype(jnp.int32), axis=1,
                      keepdims=True)
        return jnp.where(cnt >= jcol, trial, tcur)

    tbits = lax.fori_loop(0, 31, body, jnp.zeros((_B, 1), jnp.int32))
    tval = lax.bitcast_convert_type(tbits, jnp.float32)
    gt_mask = x > tval
    cnt_gt = jnp.sum(gt_mask.astype(jnp.int32), axis=1, keepdims=True)
    sum_gt = jnp.sum(jnp.where(gt_mask, x, 0.0), axis=1, keepdims=True)
    topk = jnp.where(jcol > 0,
                     sum_gt + (jcol - cnt_gt).astype(jnp.float32) * tval,
                     0.0)

    nsum = jnp.sum(npos_col)
    n = jnp.maximum(nsum, 1.0)
    loss_c = (cepos_sum + jnp.sum(topk)) / n
    loss_landm = sl1_sum / n
    loss_weight = wpen_sum / n

    ri = lax.broadcasted_iota(jnp.int32, (8, 128), 0)
    li = lax.broadcasted_iota(jnp.int32, (8, 128), 1)
    out = (jnp.where((ri == 0) & (li == 1), loss_c, 0.0)
           + jnp.where((ri == 0) & (li == 2), loss_landm, 0.0)
           + jnp.where((ri == 0) & (li == 3), loss_weight, 0.0))
    out_ref[...] = out


def kernel(loc_data, conf_data, landm_pca_weight, prior_landmarks, targets,
           pca_mean, pca_features, singular_values):
    del loc_data
    conf_t = jnp.transpose(conf_data, (0, 2, 1))
    priors_t = prior_landmarks.T
    pm_col = pca_mean.reshape(_LDIM, 1)
    sv_row = singular_values.reshape(1, _K)

    x3, acc = pl.pallas_call(
        _phase_a,
        grid=(_T, _B),
        in_specs=[
            pl.BlockSpec((1, 2, _PT), lambda t, b: (b, 0, t)),
            pl.BlockSpec((1, _PT, _K), lambda t, b: (b, t, 0)),
            pl.BlockSpec((_LDIM, _PT), lambda t, b: (0, t)),
            pl.BlockSpec((1, _G, 13), lambda t, b: (b, 0, 0)),
            pl.BlockSpec((_LDIM, 1), lambda t, b: (0, 0)),
            pl.BlockSpec((_K, _LDIM), lambda t, b: (0, 0)),
            pl.BlockSpec((1, _K), lambda t, b: (0, 0)),
        ],
        out_specs=[
            pl.BlockSpec((1, 1, _PT), lambda t, b: (b, 0, t)),
            pl.BlockSpec((_B, 128), lambda t, b: (0, 0)),
        ],
        out_shape=[
            jax.ShapeDtypeStruct((_B, 1, _P), jnp.float32),
            jax.ShapeDtypeStruct((_B, 128), jnp.float32),
        ],
    )(conf_t, landm_pca_weight, priors_t, targets, pm_col, pca_features,
      sv_row)

    out8 = pl.pallas_call(
        _phase_b,
        out_shape=jax.ShapeDtypeStruct((8, 128), jnp.float32),
    )(x3, acc)

    focal_loss = out8[0, 0]
    loss_c = out8[0, 1]
    loss_landm = out8[0, 2]
    loss_weight = out8[0, 3]
    return (focal_loss, loss_c, loss_landm, loss_weight)


# speedup vs baseline: 89.7763x; 1.2471x over previous
import jax
import jax.numpy as jnp
from jax import lax
from jax.experimental import pallas as pl

_B, _P, _G = 16, 16384, 32
_K = 32
_LDIM = 8
_SCALE = 640.0
_NEG_POS_RATIO = 7
_THRESH = 500.0
_PT = 4096
_T = _P // _PT


def _phase_a(conf_ref, lpw_ref, priors_ref, targets_ref, pm_ref, feat_ref,
             sv_ref, x_ref, acc_ref):
    t = pl.program_id(0)
    b = pl.program_id(1)

    priors_sc = priors_ref[...] * _SCALE
    tg = targets_ref[0]
    gt = tg[:, 4:4 + _LDIM] * _SCALE
    labels_col = tg[:, 12:13]

    pn_row = jnp.sum(priors_sc * priors_sc, axis=0, keepdims=True)
    gn_col = jnp.sum(gt * gt, axis=1, keepdims=True)
    cross = jnp.dot(gt, priors_sc, preferred_element_type=jnp.float32)
    d2 = pn_row + gn_col - 2.0 * cross
    d2c = jnp.maximum(d2, 0.0)
    dmin = jnp.min(d2c, axis=0, keepdims=True)
    gidx = lax.broadcasted_iota(jnp.int32, (_G, _PT), 0)
    cand = jnp.where(d2c == dmin, gidx, _G)
    bestg = jnp.min(cand, axis=0, keepdims=True)
    onehot = (gidx == bestg).astype(jnp.float32)

    matched = lax.dot_general(gt, onehot, (((0,), (0,)), ((), ())),
                              preferred_element_type=jnp.float32)
    mlabel = jnp.sum(onehot * labels_col, axis=0, keepdims=True)
    best_d = jnp.sqrt(dmin)
    posb = (best_d < _THRESH) & (mlabel > 0.0)
    posf = posb.astype(jnp.float32)
    npos_t = jnp.sum(posf)

    residual = matched - priors_sc - pm_ref[...]
    wt = lax.dot_general(residual, feat_ref[...], (((0,), (1,)), ((), ())),
                         preferred_element_type=jnp.float32)
    lpw = lpw_ref[0]
    ad = jnp.abs(lpw - wt)
    sl1 = jnp.where(ad < 1.0, 0.5 * ad * ad, ad - 0.5)
    labs = jnp.abs(lpw)
    srow = jnp.dot(posf, sl1, preferred_element_type=jnp.float32)
    wrow = jnp.dot(posf, labs, preferred_element_type=jnp.float32)
    isv_row = 1.0 / jnp.sqrt(sv_ref[...])
    sl1v = jnp.sum(srow)
    wv = jnp.sum(wrow * isv_row)

    c = conf_ref[0]
    c0 = c[0:1, :]
    c1 = c[1:2, :]
    m = jnp.maximum(c0, c1)
    lse = m + jnp.log(jnp.exp(c0 - m) + jnp.exp(c1 - m))
    cep = jnp.sum(jnp.where(posb, lse - c1, 0.0))
    x = jnp.where(posb, 0.0, lse - c0)
    x_ref[...] = x.reshape(1, 1, _PT)

    ri = lax.broadcasted_iota(jnp.int32, (_B, 128), 0)
    li = lax.broadcasted_iota(jnp.int32, (_B, 128), 1)
    vals = jnp.where(li == 0, npos_t,
                     jnp.where(li == 1, sl1v,
                               jnp.where(li == 2, wv,
                                         jnp.where(li == 3, cep, 0.0))))
    contrib = jnp.where(ri == b, vals, 0.0)
    first = (t == 0) & (b == 0)

    @pl.when(first)
    def _():
        acc_ref[...] = contrib

    @pl.when(jnp.logical_not(first))
    def _():
        acc_ref[...] = acc_ref[...] + contrib


def _phase_b(x_ref, acc_ref, out_ref):
    x = x_ref[:, 0, :]
    acc = acc_ref[...]
    npos_col = acc[:, 0:1]
    sl1_sum = jnp.sum(acc[:, 1:2])
    wpen_sum = jnp.sum(acc[:, 2:3])
    cepos_sum = jnp.sum(acc[:, 3:4])

    npos_i = npos_col.astype(jnp.int32)
    jcol = jnp.minimum(_NEG_POS_RATIO * npos_i, _P - npos_i)
    bits = lax.bitcast_convert_type(x, jnp.int32)

    def body(i, tcur):
        bit = 30 - i
        trial = jnp.bitwise_or(tcur, jnp.left_shift(jnp.int32(1), bit))
        cnt = jnp.sum((bits >= trial).astype(jnp.int32), axis=1,
                      keepdims=True)
        return jnp.where(cnt >= jcol, trial, tcur)

    tbits = lax.fori_loop(0, 31, body, jnp.zeros((_B, 1), jnp.int32))
    tval = lax.bitcast_convert_type(tbits, jnp.float32)
    gt_mask = x > tval
    cnt_gt = jnp.sum(gt_mask.astype(jnp.int32), axis=1, keepdims=True)
    sum_gt = jnp.sum(jnp.where(gt_mask, x, 0.0), axis=1, keepdims=True)
    topk = jnp.where(jcol > 0,
                     sum_gt + (jcol - cnt_gt).astype(jnp.float32) * tval,
                     0.0)

    nsum = jnp.sum(npos_col)
    n = jnp.maximum(nsum, 1.0)
    loss_c = (cepos_sum + jnp.sum(topk)) / n
    loss_landm = sl1_sum / n
    loss_weight = wpen_sum / n

    ri = lax.broadcasted_iota(jnp.int32, (8, 128), 0)
    li = lax.broadcasted_iota(jnp.int32, (8, 128), 1)
    out = (jnp.where((ri == 0) & (li == 1), loss_c, 0.0)
           + jnp.where((ri == 0) & (li == 2), loss_landm, 0.0)
           + jnp.where((ri == 0) & (li == 3), loss_weight, 0.0))
    out_ref[...] = out


def kernel(loc_data, conf_data, landm_pca_weight, prior_landmarks, targets,
           pca_mean, pca_features, singular_values):
    del loc_data
    conf_t = jnp.transpose(conf_data, (0, 2, 1))
    priors_t = prior_landmarks.T
    pm_col = pca_mean.reshape(_LDIM, 1)
    sv_row = singular_values.reshape(1, _K)

    x3, acc = pl.pallas_call(
        _phase_a,
        grid=(_T, _B),
        in_specs=[
            pl.BlockSpec((1, 2, _PT), lambda t, b: (b, 0, t)),
            pl.BlockSpec((1, _PT, _K), lambda t, b: (b, t, 0)),
            pl.BlockSpec((_LDIM, _PT), lambda t, b: (0, t)),
            pl.BlockSpec((1, _G, 13), lambda t, b: (b, 0, 0)),
            pl.BlockSpec((_LDIM, 1), lambda t, b: (0, 0)),
            pl.BlockSpec((_K, _LDIM), lambda t, b: (0, 0)),
            pl.BlockSpec((1, _K), lambda t, b: (0, 0)),
        ],
        out_specs=[
            pl.BlockSpec((1, 1, _PT), lambda t, b: (b, 0, t)),
            pl.BlockSpec((_B, 128), lambda t, b: (0, 0)),
        ],
        out_shape=[
            jax.ShapeDtypeStruct((_B, 1, _P), jnp.float32),
            jax.ShapeDtypeStruct((_B, 128), jnp.float32),
        ],
    )(conf_t, landm_pca_weight, priors_t, targets, pm_col, pca_features,
      sv_row)

    out8 = pl.pallas_call(
        _phase_b,
        out_shape=jax.ShapeDtypeStruct((8, 128), jnp.float32),
    )(x3, acc)

    focal_loss = out8[0, 0]
    loss_c = out8[0, 1]
    loss_landm = out8[0, 2]
    loss_weight = out8[0, 3]
    return (focal_loss, loss_c, loss_landm, loss_weight)


# speedup vs baseline: 99.3119x; 1.1062x over previous
import jax
import jax.numpy as jnp
from jax import lax
from jax.experimental import pallas as pl

_B, _P, _G = 16, 16384, 32
_K = 32
_LDIM = 8
_SCALE = 640.0
_NEG_POS_RATIO = 7
_THRESH = 500.0
_PT = 8192
_T = _P // _PT


def _phase_a(conf_ref, lpw_ref, priors_ref, targets_ref, pm_ref, feat_ref,
             sv_ref, x_ref, acc_ref):
    t = pl.program_id(0)
    b = pl.program_id(1)

    priors_sc = priors_ref[...] * _SCALE
    tg = targets_ref[0]
    gt = tg[:, 4:4 + _LDIM] * _SCALE
    labels_col = tg[:, 12:13]

    pn_row = jnp.sum(priors_sc * priors_sc, axis=0, keepdims=True)
    gn_col = jnp.sum(gt * gt, axis=1, keepdims=True)
    cross = jnp.dot(gt, priors_sc, preferred_element_type=jnp.float32)
    d2 = pn_row + gn_col - 2.0 * cross
    d2c = jnp.maximum(d2, 0.0)
    dmin = jnp.min(d2c, axis=0, keepdims=True)
    gidx = lax.broadcasted_iota(jnp.int32, (_G, _PT), 0)
    cand = jnp.where(d2c == dmin, gidx, _G)
    bestg = jnp.min(cand, axis=0, keepdims=True)
    onehot = (gidx == bestg).astype(jnp.float32)

    matched = lax.dot_general(gt, onehot, (((0,), (0,)), ((), ())),
                              preferred_element_type=jnp.float32)
    mlabel = jnp.sum(onehot * labels_col, axis=0, keepdims=True)
    best_d = jnp.sqrt(dmin)
    posb = (best_d < _THRESH) & (mlabel > 0.0)
    posf = posb.astype(jnp.float32)
    npos_t = jnp.sum(posf)

    residual = matched - priors_sc - pm_ref[...]
    wt = lax.dot_general(residual, feat_ref[...], (((0,), (1,)), ((), ())),
                         preferred_element_type=jnp.float32)
    lpw = lpw_ref[0]
    ad = jnp.abs(lpw - wt)
    sl1 = jnp.where(ad < 1.0, 0.5 * ad * ad, ad - 0.5)
    labs = jnp.abs(lpw)
    srow = jnp.dot(posf, sl1, preferred_element_type=jnp.float32)
    wrow = jnp.dot(posf, labs, preferred_element_type=jnp.float32)
    isv_row = 1.0 / jnp.sqrt(sv_ref[...])
    sl1v = jnp.sum(srow)
    wv = jnp.sum(wrow * isv_row)

    c = conf_ref[0]
    c0 = c[0:1, :]
    c1 = c[1:2, :]
    m = jnp.maximum(c0, c1)
    lse = m + jnp.log(jnp.exp(c0 - m) + jnp.exp(c1 - m))
    cep = jnp.sum(jnp.where(posb, lse - c1, 0.0))
    x = jnp.where(posb, 0.0, lse - c0)
    x_ref[...] = x.reshape(1, 1, _PT)

    ri = lax.broadcasted_iota(jnp.int32, (_B, 128), 0)
    li = lax.broadcasted_iota(jnp.int32, (_B, 128), 1)
    vals = jnp.where(li == 0, npos_t,
                     jnp.where(li == 1, sl1v,
                               jnp.where(li == 2, wv,
                                         jnp.where(li == 3, cep, 0.0))))
    contrib = jnp.where(ri == b, vals, 0.0)
    first = (t == 0) & (b == 0)

    @pl.when(first)
    def _():
        acc_ref[...] = contrib

    @pl.when(jnp.logical_not(first))
    def _():
        acc_ref[...] = acc_ref[...] + contrib


def _phase_b(x_ref, acc_ref, out_ref):
    x = x_ref[:, 0, :]
    acc = acc_ref[...]
    npos_col = acc[:, 0:1]
    sl1_sum = jnp.sum(acc[:, 1:2])
    wpen_sum = jnp.sum(acc[:, 2:3])
    cepos_sum = jnp.sum(acc[:, 3:4])

    npos_i = npos_col.astype(jnp.int32)
    jcol = jnp.minimum(_NEG_POS_RATIO * npos_i, _P - npos_i)
    bits = lax.bitcast_convert_type(x, jnp.int32)

    def body(i, tcur):
        bit = 30 - i
        trial = jnp.bitwise_or(tcur, jnp.left_shift(jnp.int32(1), bit))
        cnt = jnp.sum((bits >= trial).astype(jnp.int32), axis=1,
                      keepdims=True)
        return jnp.where(cnt >= jcol, trial, tcur)

    tbits = lax.fori_loop(0, 31, body, jnp.zeros((_B, 1), jnp.int32))
    tval = lax.bitcast_convert_type(tbits, jnp.float32)
    gt_mask = x > tval
    cnt_gt = jnp.sum(gt_mask.astype(jnp.int32), axis=1, keepdims=True)
    sum_gt = jnp.sum(jnp.where(gt_mask, x, 0.0), axis=1, keepdims=True)
    topk = jnp.where(jcol > 0,
                     sum_gt + (jcol - cnt_gt).astype(jnp.float32) * tval,
                     0.0)

    nsum = jnp.sum(npos_col)
    n = jnp.maximum(nsum, 1.0)
    loss_c = (cepos_sum + jnp.sum(topk)) / n
    loss_landm = sl1_sum / n
    loss_weight = wpen_sum / n

    ri = lax.broadcasted_iota(jnp.int32, (8, 128), 0)
    li = lax.broadcasted_iota(jnp.int32, (8, 128), 1)
    out = (jnp.where((ri == 0) & (li == 1), loss_c, 0.0)
           + jnp.where((ri == 0) & (li == 2), loss_landm, 0.0)
           + jnp.where((ri == 0) & (li == 3), loss_weight, 0.0))
    out_ref[...] = out


def kernel(loc_data, conf_data, landm_pca_weight, prior_landmarks, targets,
           pca_mean, pca_features, singular_values):
    del loc_data
    conf_t = jnp.transpose(conf_data, (0, 2, 1))
    priors_t = prior_landmarks.T
    pm_col = pca_mean.reshape(_LDIM, 1)
    sv_row = singular_values.reshape(1, _K)

    x3, acc = pl.pallas_call(
        _phase_a,
        grid=(_T, _B),
        in_specs=[
            pl.BlockSpec((1, 2, _PT), lambda t, b: (b, 0, t)),
            pl.BlockSpec((1, _PT, _K), lambda t, b: (b, t, 0)),
            pl.BlockSpec((_LDIM, _PT), lambda t, b: (0, t)),
            pl.BlockSpec((1, _G, 13), lambda t, b: (b, 0, 0)),
            pl.BlockSpec((_LDIM, 1), lambda t, b: (0, 0)),
            pl.BlockSpec((_K, _LDIM), lambda t, b: (0, 0)),
            pl.BlockSpec((1, _K), lambda t, b: (0, 0)),
        ],
        out_specs=[
            pl.BlockSpec((1, 1, _PT), lambda t, b: (b, 0, t)),
            pl.BlockSpec((_B, 128), lambda t, b: (0, 0)),
        ],
        out_shape=[
            jax.ShapeDtypeStruct((_B, 1, _P), jnp.float32),
            jax.ShapeDtypeStruct((_B, 128), jnp.float32),
        ],
    )(conf_t, landm_pca_weight, priors_t, targets, pm_col, pca_features,
      sv_row)

    out8 = pl.pallas_call(
        _phase_b,
        out_shape=jax.ShapeDtypeStruct((8, 128), jnp.float32),
    )(x3, acc)

    focal_loss = out8[0, 0]
    loss_c = out8[0, 1]
    loss_landm = out8[0, 2]
    loss_weight = out8[0, 3]
    return (focal_loss, loss_c, loss_landm, loss_weight)


# speedup vs baseline: 103.9995x; 1.0472x over previous
import jax
import jax.numpy as jnp
from jax import lax
from jax.experimental import pallas as pl

_B, _P, _G = 16, 16384, 32
_K = 32
_LDIM = 8
_SCALE = 640.0
_NEG_POS_RATIO = 7
_THRESH = 500.0
_PT = 16384
_T = _P // _PT


def _phase_a(conf_ref, lpw_ref, priors_ref, targets_ref, pm_ref, feat_ref,
             sv_ref, x_ref, acc_ref):
    t = pl.program_id(0)
    b = pl.program_id(1)

    priors_sc = priors_ref[...] * _SCALE
    tg = targets_ref[0]
    gt = tg[:, 4:4 + _LDIM] * _SCALE
    labels_col = tg[:, 12:13]

    pn_row = jnp.sum(priors_sc * priors_sc, axis=0, keepdims=True)
    gn_col = jnp.sum(gt * gt, axis=1, keepdims=True)
    cross = jnp.dot(gt, priors_sc, preferred_element_type=jnp.float32)
    d2 = pn_row + gn_col - 2.0 * cross
    d2c = jnp.maximum(d2, 0.0)
    dmin = jnp.min(d2c, axis=0, keepdims=True)
    gidx = lax.broadcasted_iota(jnp.int32, (_G, _PT), 0)
    cand = jnp.where(d2c == dmin, gidx, _G)
    bestg = jnp.min(cand, axis=0, keepdims=True)
    onehot = (gidx == bestg).astype(jnp.float32)

    matched = lax.dot_general(gt, onehot, (((0,), (0,)), ((), ())),
                              preferred_element_type=jnp.float32)
    mlabel = jnp.sum(onehot * labels_col, axis=0, keepdims=True)
    best_d = jnp.sqrt(dmin)
    posb = (best_d < _THRESH) & (mlabel > 0.0)
    posf = posb.astype(jnp.float32)
    npos_t = jnp.sum(posf)

    residual = matched - priors_sc - pm_ref[...]
    wt = lax.dot_general(residual, feat_ref[...], (((0,), (1,)), ((), ())),
                         preferred_element_type=jnp.float32)
    lpw = lpw_ref[0]
    ad = jnp.abs(lpw - wt)
    sl1 = jnp.where(ad < 1.0, 0.5 * ad * ad, ad - 0.5)
    labs = jnp.abs(lpw)
    srow = jnp.dot(posf, sl1, preferred_element_type=jnp.float32)
    wrow = jnp.dot(posf, labs, preferred_element_type=jnp.float32)
    isv_row = 1.0 / jnp.sqrt(sv_ref[...])
    sl1v = jnp.sum(srow)
    wv = jnp.sum(wrow * isv_row)

    c = conf_ref[0]
    c0 = c[0:1, :]
    c1 = c[1:2, :]
    m = jnp.maximum(c0, c1)
    lse = m + jnp.log(jnp.exp(c0 - m) + jnp.exp(c1 - m))
    cep = jnp.sum(jnp.where(posb, lse - c1, 0.0))
    x = jnp.where(posb, 0.0, lse - c0)
    x_ref[...] = x.reshape(1, 1, _PT)

    ri = lax.broadcasted_iota(jnp.int32, (_B, 128), 0)
    li = lax.broadcasted_iota(jnp.int32, (_B, 128), 1)
    vals = jnp.where(li == 0, npos_t,
                     jnp.where(li == 1, sl1v,
                               jnp.where(li == 2, wv,
                                         jnp.where(li == 3, cep, 0.0))))
    contrib = jnp.where(ri == b, vals, 0.0)
    first = (t == 0) & (b == 0)

    @pl.when(first)
    def _():
        acc_ref[...] = contrib

    @pl.when(jnp.logical_not(first))
    def _():
        acc_ref[...] = acc_ref[...] + contrib


def _phase_b(x_ref, acc_ref, out_ref):
    x = x_ref[:, 0, :]
    acc = acc_ref[...]
    npos_col = acc[:, 0:1]
    sl1_sum = jnp.sum(acc[:, 1:2])
    wpen_sum = jnp.sum(acc[:, 2:3])
    cepos_sum = jnp.sum(acc[:, 3:4])

    npos_i = npos_col.astype(jnp.int32)
    jcol = jnp.minimum(_NEG_POS_RATIO * npos_i, _P - npos_i)
    bits = lax.bitcast_convert_type(x, jnp.int32)

    def body(i, tcur):
        bit = 30 - i
        trial = jnp.bitwise_or(tcur, jnp.left_shift(jnp.int32(1), bit))
        cnt = jnp.sum((bits >= trial).astype(jnp.int32), axis=1,
                      keepdims=True)
        return jnp.where(cnt >= jcol, trial, tcur)

    tbits = lax.fori_loop(0, 31, body, jnp.zeros((_B, 1), jnp.int32))
    tval = lax.bitcast_convert_type(tbits, jnp.float32)
    gt_mask = x > tval
    cnt_gt = jnp.sum(gt_mask.astype(jnp.int32), axis=1, keepdims=True)
    sum_gt = jnp.sum(jnp.where(gt_mask, x, 0.0), axis=1, keepdims=True)
    topk = jnp.where(jcol > 0,
                     sum_gt + (jcol - cnt_gt).astype(jnp.float32) * tval,
                     0.0)

    nsum = jnp.sum(npos_col)
    n = jnp.maximum(nsum, 1.0)
    loss_c = (cepos_sum + jnp.sum(topk)) / n
    loss_landm = sl1_sum / n
    loss_weight = wpen_sum / n

    ri = lax.broadcasted_iota(jnp.int32, (8, 128), 0)
    li = lax.broadcasted_iota(jnp.int32, (8, 128), 1)
    out = (jnp.where((ri == 0) & (li == 1), loss_c, 0.0)
           + jnp.where((ri == 0) & (li == 2), loss_landm, 0.0)
           + jnp.where((ri == 0) & (li == 3), loss_weight, 0.0))
    out_ref[...] = out


def kernel(loc_data, conf_data, landm_pca_weight, prior_landmarks, targets,
           pca_mean, pca_features, singular_values):
    del loc_data
    conf_t = jnp.transpose(conf_data, (0, 2, 1))
    priors_t = prior_landmarks.T
    pm_col = pca_mean.reshape(_LDIM, 1)
    sv_row = singular_values.reshape(1, _K)

    x3, acc = pl.pallas_call(
        _phase_a,
        grid=(_T, _B),
        in_specs=[
            pl.BlockSpec((1, 2, _PT), lambda t, b: (b, 0, t)),
            pl.BlockSpec((1, _PT, _K), lambda t, b: (b, t, 0)),
            pl.BlockSpec((_LDIM, _PT), lambda t, b: (0, t)),
            pl.BlockSpec((1, _G, 13), lambda t, b: (b, 0, 0)),
            pl.BlockSpec((_LDIM, 1), lambda t, b: (0, 0)),
            pl.BlockSpec((_K, _LDIM), lambda t, b: (0, 0)),
            pl.BlockSpec((1, _K), lambda t, b: (0, 0)),
        ],
        out_specs=[
            pl.BlockSpec((1, 1, _PT), lambda t, b: (b, 0, t)),
            pl.BlockSpec((_B, 128), lambda t, b: (0, 0)),
        ],
        out_shape=[
            jax.ShapeDtypeStruct((_B, 1, _P), jnp.float32),
            jax.ShapeDtypeStruct((_B, 128), jnp.float32),
        ],
    )(conf_t, landm_pca_weight, priors_t, targets, pm_col, pca_features,
      sv_row)

    out8 = pl.pallas_call(
        _phase_b,
        out_shape=jax.ShapeDtypeStruct((8, 128), jnp.float32),
    )(x3, acc)

    focal_loss = out8[0, 0]
    loss_c = out8[0, 1]
    loss_landm = out8[0, 2]
    loss_weight = out8[0, 3]
    return (focal_loss, loss_c, loss_landm, loss_weight)


# speedup vs baseline: 104.1011x; 1.0010x over previous
import jax
import jax.numpy as jnp
from jax import lax
from jax.experimental import pallas as pl
from jax.experimental.pallas import tpu as pltpu

_B, _P, _G = 16, 16384, 32
_K = 32
_LDIM = 8
_SCALE = 640.0
_NEG_POS_RATIO = 7
_THRESH = 500.0
_PT = 16384
_T = _P // _PT


def _phase_a(conf_ref, lpw_ref, priors_ref, targets_ref, pm_ref, feat_ref,
             sv_ref, x_ref, acc_ref):
    priors_sc = priors_ref[...] * _SCALE
    tg = targets_ref[0]
    gt = tg[:, 4:4 + _LDIM] * _SCALE
    labels_col = tg[:, 12:13]

    pn_row = jnp.sum(priors_sc * priors_sc, axis=0, keepdims=True)
    gn_col = jnp.sum(gt * gt, axis=1, keepdims=True)
    cross = jnp.dot(gt, priors_sc, preferred_element_type=jnp.float32)
    d2 = pn_row + gn_col - 2.0 * cross
    d2c = jnp.maximum(d2, 0.0)
    dmin = jnp.min(d2c, axis=0, keepdims=True)
    gidx = lax.broadcasted_iota(jnp.int32, (_G, _PT), 0)
    cand = jnp.where(d2c == dmin, gidx, _G)
    bestg = jnp.min(cand, axis=0, keepdims=True)
    onehot = (gidx == bestg).astype(jnp.float32)

    matched = lax.dot_general(gt, onehot, (((0,), (0,)), ((), ())),
                              preferred_element_type=jnp.float32)
    mlabel = jnp.sum(onehot * labels_col, axis=0, keepdims=True)
    best_d = jnp.sqrt(dmin)
    posb = (best_d < _THRESH) & (mlabel > 0.0)
    posf = posb.astype(jnp.float32)
    npos_t = jnp.sum(posf)

    residual = matched - priors_sc - pm_ref[...]
    wt = lax.dot_general(residual, feat_ref[...], (((0,), (1,)), ((), ())),
                         preferred_element_type=jnp.float32)
    lpw = lpw_ref[0]
    ad = jnp.abs(lpw - wt)
    sl1 = jnp.where(ad < 1.0, 0.5 * ad * ad, ad - 0.5)
    labs = jnp.abs(lpw)
    srow = jnp.dot(posf, sl1, preferred_element_type=jnp.float32)
    wrow = jnp.dot(posf, labs, preferred_element_type=jnp.float32)
    isv_row = 1.0 / jnp.sqrt(sv_ref[...])
    sl1v = jnp.sum(srow)
    wv = jnp.sum(wrow * isv_row)

    c = conf_ref[0]
    c0 = c[0:1, :]
    c1 = c[1:2, :]
    m = jnp.maximum(c0, c1)
    lse = m + jnp.log(jnp.exp(c0 - m) + jnp.exp(c1 - m))
    cep = jnp.sum(jnp.where(posb, lse - c1, 0.0))
    x = jnp.where(posb, 0.0, lse - c0)
    x_ref[...] = x.reshape(1, 1, _PT)

    li = lax.broadcasted_iota(jnp.int32, (1, 1, 128), 2)
    vals = jnp.where(li == 0, npos_t,
                     jnp.where(li == 1, sl1v,
                               jnp.where(li == 2, wv,
                                         jnp.where(li == 3, cep, 0.0))))
    acc_ref[...] = vals


def _phase_b(x_ref, acc_ref, out_ref):
    x = x_ref[:, 0, :]
    acc = acc_ref[:, 0, :]
    npos_col = acc[:, 0:1]
    sl1_sum = jnp.sum(acc[:, 1:2])
    wpen_sum = jnp.sum(acc[:, 2:3])
    cepos_sum = jnp.sum(acc[:, 3:4])

    npos_i = npos_col.astype(jnp.int32)
    jcol = jnp.minimum(_NEG_POS_RATIO * npos_i, _P - npos_i)
    bits = lax.bitcast_convert_type(x, jnp.int32)

    def body(i, tcur):
        bit = 30 - i
        trial = jnp.bitwise_or(tcur, jnp.left_shift(jnp.int32(1), bit))
        cnt = jnp.sum((bits >= trial).astype(jnp.int32), axis=1,
                      keepdims=True)
        return jnp.where(cnt >= jcol, trial, tcur)

    tbits = lax.fori_loop(0, 31, body, jnp.zeros((_B, 1), jnp.int32))
    tval = lax.bitcast_convert_type(tbits, jnp.float32)
    gt_mask = x > tval
    cnt_gt = jnp.sum(gt_mask.astype(jnp.int32), axis=1, keepdims=True)
    sum_gt = jnp.sum(jnp.where(gt_mask, x, 0.0), axis=1, keepdims=True)
    topk = jnp.where(jcol > 0,
                     sum_gt + (jcol - cnt_gt).astype(jnp.float32) * tval,
                     0.0)

    nsum = jnp.sum(npos_col)
    n = jnp.maximum(nsum, 1.0)
    loss_c = (cepos_sum + jnp.sum(topk)) / n
    loss_landm = sl1_sum / n
    loss_weight = wpen_sum / n

    ri = lax.broadcasted_iota(jnp.int32, (8, 128), 0)
    li = lax.broadcasted_iota(jnp.int32, (8, 128), 1)
    out = (jnp.where((ri == 0) & (li == 1), loss_c, 0.0)
           + jnp.where((ri == 0) & (li == 2), loss_landm, 0.0)
           + jnp.where((ri == 0) & (li == 3), loss_weight, 0.0))
    out_ref[...] = out


def kernel(loc_data, conf_data, landm_pca_weight, prior_landmarks, targets,
           pca_mean, pca_features, singular_values):
    del loc_data
    conf_t = jnp.transpose(conf_data, (0, 2, 1))
    priors_t = prior_landmarks.T
    pm_col = pca_mean.reshape(_LDIM, 1)
    sv_row = singular_values.reshape(1, _K)

    x3, acc = pl.pallas_call(
        _phase_a,
        grid=(_B,),
        in_specs=[
            pl.BlockSpec((1, 2, _PT), lambda b: (b, 0, 0)),
            pl.BlockSpec((1, _PT, _K), lambda b: (b, 0, 0)),
            pl.BlockSpec((_LDIM, _PT), lambda b: (0, 0)),
            pl.BlockSpec((1, _G, 13), lambda b: (b, 0, 0)),
            pl.BlockSpec((_LDIM, 1), lambda b: (0, 0)),
            pl.BlockSpec((_K, _LDIM), lambda b: (0, 0)),
            pl.BlockSpec((1, _K), lambda b: (0, 0)),
        ],
        out_specs=[
            pl.BlockSpec((1, 1, _PT), lambda b: (b, 0, 0)),
            pl.BlockSpec((1, 1, 128), lambda b: (b, 0, 0)),
        ],
        out_shape=[
            jax.ShapeDtypeStruct((_B, 1, _P), jnp.float32),
            jax.ShapeDtypeStruct((_B, 1, 128), jnp.float32),
        ],
        compiler_params=pltpu.CompilerParams(
            dimension_semantics=("parallel",)),
    )(conf_t, landm_pca_weight, priors_t, targets, pm_col, pca_features,
      sv_row)

    out8 = pl.pallas_call(
        _phase_b,
        out_shape=jax.ShapeDtypeStruct((8, 128), jnp.float32),
    )(x3, acc)

    focal_loss = out8[0, 0]
    loss_c = out8[0, 1]
    loss_landm = out8[0, 2]
    loss_weight = out8[0, 3]
    return (focal_loss, loss_c, loss_landm, loss_weight)
